# Initial kernel scaffold; baseline (speedup 1.0000x reference)
import jax
import jax.numpy as jnp
from jax.experimental import pallas as pl


def kernel(x, Wg, W1, b1, W2, b2):
    raise NotImplementedError("write your pallas kernel here")


# speedup vs baseline: 1.5933x; 1.5933x over previous
import functools

import jax
import jax.numpy as jnp
from jax import lax
from jax.experimental import pallas as pl
from jax.experimental.pallas import tpu as pltpu
from jax.experimental.pallas import tpu_sc as plsc

BLK = 128
LANES = 16

_SC_PARAMS = pltpu.CompilerParams(needs_layout_passes=False)
_DEBUG_JNP_SCHEDULE = False


def _router_body(x_ref, wg_ref, eid_ref):
    logits = jnp.dot(x_ref[...], wg_ref[...],
                     preferred_element_type=jnp.float32)
    mx = jnp.max(logits, axis=1, keepdims=True)
    ii = lax.broadcasted_iota(jnp.int32, logits.shape, 1)
    big = jnp.int32(logits.shape[1])
    eid_ref[...] = jnp.min(jnp.where(logits == mx, ii, big), axis=1,
                           keepdims=True)


def _router(xr, Wg):
    T, _ = xr.shape
    return pl.pallas_call(
        _router_body,
        out_shape=jax.ShapeDtypeStruct((T, 1), jnp.int32),
    )(xr, Wg)


def _schedule_body(T, E, NB, P, eid_hbm, src_hbm, be_hbm, lv_hbm, dbg_hbm,
                   eid_v, buf_v, cnt_v, pad_v, be_v, lv_v, csh, crow_v):
    cid = lax.axis_index("core")
    sid = lax.axis_index("subcore")
    lane = lax.iota(jnp.int32, LANES)
    padmark = jnp.full((LANES,), T, jnp.int32)

    @pl.when(cid == 0)
    def _():
        e = sid
        pltpu.sync_copy(eid_hbm, eid_v)

        @pl.loop(0, buf_v.shape[0], step=LANES)
        def _(i):
            buf_v[pl.ds(i, LANES)] = padmark

        def scan(j, carry):
            off, counts = carry
            v = eid_v[pl.ds(j * LANES, LANES)]
            m = v == e
            tok = lane + j * LANES
            plsc.store_compressed(buf_v.at[pl.ds(off, LANES)], tok, mask=m)
            off = off + jnp.max(plsc.all_reduce_population_count(m))
            for e2 in range(E):
                pc = plsc.all_reduce_population_count(v == e2)
                counts = counts + jnp.where(lane == e2, pc, 0)
            return off, counts

        _, counts = lax.fori_loop(0, T // LANES, scan,
                                  (jnp.int32(0), jnp.zeros((LANES,), jnp.int32)))

        cnt_v[...] = counts
        pltpu.sync_copy(cnt_v, dbg_hbm.at[sid])

        nb = (counts + (BLK - 1)) // BLK
        cum = plsc.cumsum(nb)
        start = (cum - nb) * BLK
        my = lane == e
        p_e = jnp.max(jnp.where(my, start, 0))
        nb_e = jnp.max(jnp.where(my, nb, 0))

        def cp(k, carry):
            off = pl.multiple_of(p_e + k * BLK, BLK)
            pltpu.sync_copy(buf_v.at[pl.ds(k * BLK, BLK)],
                            src_hbm.at[pl.ds(off, BLK)])
            return carry

        lax.fori_loop(0, nb_e, cp, jnp.int32(0))

        @pl.when(sid == 0)
        def _():
            total = jnp.max(jnp.where(lane == E - 1, cum, 0))

            @pl.loop(0, BLK, step=LANES)
            def _(i):
                pad_v[pl.ds(i, LANES)] = padmark

            def tail(b, carry):
                off = pl.multiple_of(b * BLK, BLK)
                pltpu.sync_copy(pad_v, src_hbm.at[pl.ds(off, BLK)])
                return carry

            lax.fori_loop(total, NB, tail, jnp.int32(0))

            for g in range(NB // LANES):
                bvec = lane + g * LANES
                bc = jnp.minimum(bvec, total - 1)
                acc = jnp.zeros((LANES,), jnp.int32)
                for e2 in range(E):
                    ce2 = jnp.max(jnp.where(lane == e2, cum, 0))
                    acc = acc + (bc >= ce2).astype(jnp.int32)
                be_v[pl.ds(g * LANES, LANES)] = acc
                lv_v[pl.ds(g * LANES, LANES)] = (bvec < total).astype(jnp.int32)
            pltpu.sync_copy(be_v, be_hbm)
            pltpu.sync_copy(lv_v, lv_hbm)


def _schedule(eid, E, NB, P):
    (T,) = eid.shape
    mesh = plsc.VectorSubcoreMesh(core_axis_name="core",
                                  subcore_axis_name="subcore",
                                  num_cores=2, num_subcores=16)
    k = pl.kernel(
        functools.partial(_schedule_body, T, E, NB, P),
        out_type=(jax.ShapeDtypeStruct((P,), jnp.int32),
                  jax.ShapeDtypeStruct((NB,), jnp.int32),
                  jax.ShapeDtypeStruct((NB,), jnp.int32),
                  jax.ShapeDtypeStruct((LANES, LANES), jnp.int32)),
        mesh=mesh,
        scratch_types=[
            pltpu.VMEM((T,), jnp.int32),
            pltpu.VMEM((T + BLK,), jnp.int32),
            pltpu.VMEM((LANES,), jnp.int32),
            pltpu.VMEM((BLK,), jnp.int32),
            pltpu.VMEM((NB,), jnp.int32),
            pltpu.VMEM((NB,), jnp.int32),
            pltpu.VMEM_SHARED((LANES, LANES), jnp.int32),
            pltpu.VMEM((LANES, LANES), jnp.int32),
        ],
        compiler_params=_SC_PARAMS,
    )
    return k(eid)


def _gather_body(C, T, xr_hbm, src_hbm, xs_hbm, idx_v, rows_v):
    wid = lax.axis_index("subcore") * 2 + lax.axis_index("core")
    base = pl.multiple_of(wid * C, C)
    pltpu.sync_copy(src_hbm.at[pl.ds(base, C)], idx_v)

    @pl.loop(0, C, step=LANES)
    def _(i):
        idx_v[pl.ds(i, LANES)] = jnp.minimum(idx_v[pl.ds(i, LANES)], T - 1)

    pltpu.sync_copy(xr_hbm.at[idx_v], rows_v)
    pltpu.sync_copy(rows_v, xs_hbm.at[pl.ds(base, C)])


def _gather(xr, src, P):
    T, D = xr.shape
    C = P // 32
    mesh = plsc.VectorSubcoreMesh(core_axis_name="core",
                                  subcore_axis_name="subcore",
                                  num_cores=2, num_subcores=16)
    k = pl.kernel(
        functools.partial(_gather_body, C, T),
        out_type=jax.ShapeDtypeStruct((P, D), jnp.float32),
        mesh=mesh,
        scratch_types=[
            pltpu.VMEM((C,), jnp.int32),
            pltpu.VMEM((C, D), jnp.float32),
        ],
    )
    return k(xr, src)


def _scatter_body(C, ffn_hbm, src_hbm, out_hbm, idx_v, rows_v):
    wid = lax.axis_index("subcore") * 2 + lax.axis_index("core")
    base = pl.multiple_of(wid * C, C)
    pltpu.sync_copy(src_hbm.at[pl.ds(base, C)], idx_v)
    pltpu.sync_copy(ffn_hbm.at[pl.ds(base, C)], rows_v)
    pltpu.sync_copy(rows_v, out_hbm.at[idx_v])


def _scatter(ffn_out, src, T):
    P, D = ffn_out.shape
    C = P // 32
    mesh = plsc.VectorSubcoreMesh(core_axis_name="core",
                                  subcore_axis_name="subcore",
                                  num_cores=2, num_subcores=16)
    k = pl.kernel(
        functools.partial(_scatter_body, C),
        out_type=jax.ShapeDtypeStruct((T + 8, D), jnp.float32),
        mesh=mesh,
        scratch_types=[
            pltpu.VMEM((C,), jnp.int32),
            pltpu.VMEM((C, D), jnp.float32),
        ],
    )
    return k(ffn_out, src)


def _ffn_body(be_ref, lv_ref, xs_ref, w1_ref, b1_ref, w2_ref, b2_ref, o_ref):
    i = pl.program_id(0)

    @pl.when(lv_ref[i] > 0)
    def _():
        h = jnp.dot(xs_ref[...], w1_ref[0],
                    preferred_element_type=jnp.float32) + b1_ref[0]
        h = jnp.maximum(h, 0.0)
        o_ref[...] = jnp.dot(h, w2_ref[0],
                             preferred_element_type=jnp.float32) + b2_ref[0]


def _ffn(be, lv, xs, W1, b1, W2, b2, NB):
    P, D = xs.shape
    H = W1.shape[2]
    grid_spec = pltpu.PrefetchScalarGridSpec(
        num_scalar_prefetch=2,
        grid=(NB,),
        in_specs=[
            pl.BlockSpec((BLK, D), lambda i, be, lv: (i, 0)),
            pl.BlockSpec((1, D, H), lambda i, be, lv: (be[i], 0, 0)),
            pl.BlockSpec((1, 1, H), lambda i, be, lv: (be[i], 0, 0)),
            pl.BlockSpec((1, H, D), lambda i, be, lv: (be[i], 0, 0)),
            pl.BlockSpec((1, 1, D), lambda i, be, lv: (be[i], 0, 0)),
        ],
        out_specs=pl.BlockSpec((BLK, D), lambda i, be, lv: (i, 0)),
    )
    return pl.pallas_call(
        _ffn_body,
        grid_spec=grid_spec,
        out_shape=jax.ShapeDtypeStruct((P, D), jnp.float32),
    )(be, lv, xs, W1, b1[:, None, :], W2, b2[:, None, :])


def kernel(x, Wg, W1, b1, W2, b2):
    B, S, D = x.shape
    E = Wg.shape[1]
    T = B * S
    NB = T // BLK + E
    P = NB * BLK

    xr = x.reshape(T, D)
    eid = _router(xr, Wg).reshape(T)
    if _DEBUG_JNP_SCHEDULE:
        counts = jnp.sum(eid[:, None] == jnp.arange(E)[None, :], axis=0)
        nb = (counts + BLK - 1) // BLK
        cum = jnp.cumsum(nb)
        start = (cum - nb) * BLK
        total = cum[E - 1]
        order = jnp.argsort(eid, stable=True)
        sc_ = jnp.cumsum(counts) - counts
        es = eid[order]
        slot = start[es] + (jnp.arange(T) - sc_[es])
        src = jnp.full((P,), T, jnp.int32).at[slot].set(
            order.astype(jnp.int32))
        bvec = jnp.arange(NB)
        bc = jnp.minimum(bvec, total - 1)
        be = jnp.sum(cum[None, :] <= bc[:, None], axis=1).astype(jnp.int32)
        lv = (bvec < total).astype(jnp.int32)
    else:
        src, be, lv, _unused = _schedule(eid, E, NB, P)
    xs = _gather(xr, src, P)
    out_s = _ffn(be, lv, xs, W1, b1, W2, b2, NB)
    opad = _scatter(out_s, src, T)
    return opad[:T].reshape(B, S, D)


# speedup vs baseline: 3.3327x; 2.0917x over previous
import functools

import jax
import jax.numpy as jnp
from jax import lax
from jax.experimental import pallas as pl
from jax.experimental.pallas import tpu as pltpu
from jax.experimental.pallas import tpu_sc as plsc

BLK = 128
LANES = 16

_SC_PARAMS = pltpu.CompilerParams(needs_layout_passes=False)
_DEBUG_JNP_SCHEDULE = False


def _router_body(x_ref, wg_ref, eid_ref):
    logits = jnp.dot(x_ref[...], wg_ref[...],
                     preferred_element_type=jnp.float32)
    mx = jnp.max(logits, axis=1, keepdims=True)
    ii = lax.broadcasted_iota(jnp.int32, logits.shape, 1)
    big = jnp.int32(logits.shape[1])
    eid_ref[...] = jnp.min(jnp.where(logits == mx, ii, big), axis=1,
                           keepdims=True)


def _router(xr, Wg):
    T, _ = xr.shape
    return pl.pallas_call(
        _router_body,
        out_shape=jax.ShapeDtypeStruct((T, 1), jnp.int32),
    )(xr, Wg)


def _schedule_body(T, E, NB, P, eid_hbm, src_hbm, be_hbm, lv_hbm, dbg_hbm,
                   eid_v, buf_v, cnt_v, pad_v, be_v, lv_v, csh, crow_v):
    cid = lax.axis_index("core")
    sid = lax.axis_index("subcore")
    lane = lax.iota(jnp.int32, LANES)
    padmark = jnp.full((LANES,), T, jnp.int32)

    @pl.when(cid == 0)
    def _():
        e = sid
        pltpu.sync_copy(eid_hbm, eid_v)

        @pl.loop(0, buf_v.shape[0], step=LANES)
        def _(i):
            buf_v[pl.ds(i, LANES)] = padmark

        def scan(j, carry):
            off, counts = carry
            v = eid_v[pl.ds(j * LANES, LANES)]
            m = v == e
            tok = lane + j * LANES
            plsc.store_compressed(buf_v.at[pl.ds(off, LANES)], tok, mask=m)
            off = off + jnp.max(plsc.all_reduce_population_count(m))
            for e2 in range(E):
                pc = plsc.all_reduce_population_count(v == e2)
                counts = counts + jnp.where(lane == e2, pc, 0)
            return off, counts

        _, counts = lax.fori_loop(0, T // LANES, scan,
                                  (jnp.int32(0), jnp.zeros((LANES,), jnp.int32)))

        cnt_v[...] = counts
        pltpu.sync_copy(cnt_v, dbg_hbm.at[sid])

        nb = (counts + (BLK - 1)) // BLK
        cum = plsc.cumsum(nb)
        start = (cum - nb) * BLK
        my = lane == e
        p_e = jnp.max(jnp.where(my, start, 0))
        nb_e = jnp.max(jnp.where(my, nb, 0))

        def cp(k, carry):
            off = pl.multiple_of(p_e + k * BLK, BLK)
            pltpu.sync_copy(buf_v.at[pl.ds(k * BLK, BLK)],
                            src_hbm.at[pl.ds(off, BLK)])
            return carry

        lax.fori_loop(0, nb_e, cp, jnp.int32(0))

        @pl.when(sid == 0)
        def _():
            total = jnp.max(jnp.where(lane == E - 1, cum, 0))

            @pl.loop(0, BLK, step=LANES)
            def _(i):
                pad_v[pl.ds(i, LANES)] = padmark

            def tail(b, carry):
                off = pl.multiple_of(b * BLK, BLK)
                pltpu.sync_copy(pad_v, src_hbm.at[pl.ds(off, BLK)])
                return carry

            lax.fori_loop(total, NB, tail, jnp.int32(0))

            for g in range(NB // LANES):
                bvec = lane + g * LANES
                bc = jnp.minimum(bvec, total - 1)
                acc = jnp.zeros((LANES,), jnp.int32)
                for e2 in range(E):
                    ce2 = jnp.max(jnp.where(lane == e2, cum, 0))
                    acc = acc + (bc >= ce2).astype(jnp.int32)
                be_v[pl.ds(g * LANES, LANES)] = acc
                lv_v[pl.ds(g * LANES, LANES)] = (bvec < total).astype(jnp.int32)
            pltpu.sync_copy(be_v, be_hbm)
            pltpu.sync_copy(lv_v, lv_hbm)


def _schedule(eid, E, NB, P):
    (T,) = eid.shape
    mesh = plsc.VectorSubcoreMesh(core_axis_name="core",
                                  subcore_axis_name="subcore",
                                  num_cores=2, num_subcores=16)
    k = pl.kernel(
        functools.partial(_schedule_body, T, E, NB, P),
        out_type=(jax.ShapeDtypeStruct((P,), jnp.int32),
                  jax.ShapeDtypeStruct((NB,), jnp.int32),
                  jax.ShapeDtypeStruct((NB,), jnp.int32),
                  jax.ShapeDtypeStruct((LANES, LANES), jnp.int32)),
        mesh=mesh,
        scratch_types=[
            pltpu.VMEM((T,), jnp.int32),
            pltpu.VMEM((T + BLK,), jnp.int32),
            pltpu.VMEM((LANES,), jnp.int32),
            pltpu.VMEM((BLK,), jnp.int32),
            pltpu.VMEM((NB,), jnp.int32),
            pltpu.VMEM((NB,), jnp.int32),
            pltpu.VMEM_SHARED((LANES, LANES), jnp.int32),
            pltpu.VMEM((LANES, LANES), jnp.int32),
        ],
        compiler_params=_SC_PARAMS,
    )
    return k(eid)


def _gather_body(C, T, xr_hbm, src_hbm, xs_hbm, idx_v, rows_v):
    wid = lax.axis_index("subcore") * 2 + lax.axis_index("core")
    base = pl.multiple_of(wid * C, C)
    pltpu.sync_copy(src_hbm.at[pl.ds(base, C)], idx_v)

    @pl.loop(0, C, step=LANES)
    def _(i):
        idx_v[pl.ds(i, LANES)] = jnp.minimum(idx_v[pl.ds(i, LANES)], T - 1)

    pltpu.sync_copy(xr_hbm.at[idx_v], rows_v)
    pltpu.sync_copy(rows_v, xs_hbm.at[pl.ds(base, C)])


def _gather(xr, src, P):
    T, D = xr.shape
    C = P // 32
    mesh = plsc.VectorSubcoreMesh(core_axis_name="core",
                                  subcore_axis_name="subcore",
                                  num_cores=2, num_subcores=16)
    k = pl.kernel(
        functools.partial(_gather_body, C, T),
        out_type=jax.ShapeDtypeStruct((P, D), jnp.float32),
        mesh=mesh,
        scratch_types=[
            pltpu.VMEM((C,), jnp.int32),
            pltpu.VMEM((C, D), jnp.float32),
        ],
    )
    return k(xr, src)


def _scatter_body(C, ffn_hbm, src_hbm, out_hbm, idx_v, rows_v):
    wid = lax.axis_index("subcore") * 2 + lax.axis_index("core")
    base = pl.multiple_of(wid * C, C)
    pltpu.sync_copy(src_hbm.at[pl.ds(base, C)], idx_v)
    pltpu.sync_copy(ffn_hbm.at[pl.ds(base, C)], rows_v)
    pltpu.sync_copy(rows_v, out_hbm.at[idx_v])


def _scatter(ffn_out, src, T):
    P, D = ffn_out.shape
    C = P // 32
    mesh = plsc.VectorSubcoreMesh(core_axis_name="core",
                                  subcore_axis_name="subcore",
                                  num_cores=2, num_subcores=16)
    k = pl.kernel(
        functools.partial(_scatter_body, C),
        out_type=jax.ShapeDtypeStruct((T + 8, D), jnp.float32),
        mesh=mesh,
        scratch_types=[
            pltpu.VMEM((C,), jnp.int32),
            pltpu.VMEM((C, D), jnp.float32),
        ],
    )
    return k(ffn_out, src)


def _ffn_fused_body(T, src_ref, be_ref, lv_ref, xr_ref, w1_ref, b1_ref,
                    w2_ref, b2_ref, o_ref, xs_scr, os_scr):
    i = pl.program_id(0)

    @pl.when(lv_ref[i] > 0)
    def _():
        def g(r, c):
            t = jnp.minimum(src_ref[i * BLK + r], T - 1)
            xs_scr[pl.ds(r, 1), :] = xr_ref[pl.ds(t, 1), :]
            return c

        lax.fori_loop(0, BLK, g, 0, unroll=8)
        h = jnp.dot(xs_scr[...], w1_ref[0],
                    preferred_element_type=jnp.float32) + b1_ref[0]
        h = jnp.maximum(h, 0.0)
        os_scr[...] = jnp.dot(h, w2_ref[0],
                              preferred_element_type=jnp.float32) + b2_ref[0]

        def s(r, c):
            t = src_ref[i * BLK + r]
            o_ref[pl.ds(t, 1), :] = os_scr[pl.ds(r, 1), :]
            return c

        lax.fori_loop(0, BLK, s, 0, unroll=8)


def _ffn_fused(src, be, lv, xr, W1, b1, W2, b2, NB):
    T, D = xr.shape
    H = W1.shape[2]
    grid_spec = pltpu.PrefetchScalarGridSpec(
        num_scalar_prefetch=3,
        grid=(NB,),
        in_specs=[
            pl.BlockSpec((T, D), lambda i, src, be, lv: (0, 0)),
            pl.BlockSpec((1, D, H), lambda i, src, be, lv: (be[i], 0, 0)),
            pl.BlockSpec((1, 1, H), lambda i, src, be, lv: (be[i], 0, 0)),
            pl.BlockSpec((1, H, D), lambda i, src, be, lv: (be[i], 0, 0)),
            pl.BlockSpec((1, 1, D), lambda i, src, be, lv: (be[i], 0, 0)),
        ],
        out_specs=pl.BlockSpec((T + 8, D), lambda i, src, be, lv: (0, 0)),
        scratch_shapes=[
            pltpu.VMEM((BLK, D), jnp.float32),
            pltpu.VMEM((BLK, D), jnp.float32),
        ],
    )
    return pl.pallas_call(
        functools.partial(_ffn_fused_body, T),
        grid_spec=grid_spec,
        out_shape=jax.ShapeDtypeStruct((T + 8, D), jnp.float32),
    )(src, be, lv, xr, W1, b1[:, None, :], W2, b2[:, None, :])


def _ffn_body(be_ref, lv_ref, xs_ref, w1_ref, b1_ref, w2_ref, b2_ref, o_ref):
    i = pl.program_id(0)

    @pl.when(lv_ref[i] > 0)
    def _():
        h = jnp.dot(xs_ref[...], w1_ref[0],
                    preferred_element_type=jnp.float32) + b1_ref[0]
        h = jnp.maximum(h, 0.0)
        o_ref[...] = jnp.dot(h, w2_ref[0],
                             preferred_element_type=jnp.float32) + b2_ref[0]


def _ffn(be, lv, xs, W1, b1, W2, b2, NB):
    P, D = xs.shape
    H = W1.shape[2]
    grid_spec = pltpu.PrefetchScalarGridSpec(
        num_scalar_prefetch=2,
        grid=(NB,),
        in_specs=[
            pl.BlockSpec((BLK, D), lambda i, be, lv: (i, 0)),
            pl.BlockSpec((1, D, H), lambda i, be, lv: (be[i], 0, 0)),
            pl.BlockSpec((1, 1, H), lambda i, be, lv: (be[i], 0, 0)),
            pl.BlockSpec((1, H, D), lambda i, be, lv: (be[i], 0, 0)),
            pl.BlockSpec((1, 1, D), lambda i, be, lv: (be[i], 0, 0)),
        ],
        out_specs=pl.BlockSpec((BLK, D), lambda i, be, lv: (i, 0)),
    )
    return pl.pallas_call(
        _ffn_body,
        grid_spec=grid_spec,
        out_shape=jax.ShapeDtypeStruct((P, D), jnp.float32),
    )(be, lv, xs, W1, b1[:, None, :], W2, b2[:, None, :])


def kernel(x, Wg, W1, b1, W2, b2):
    B, S, D = x.shape
    E = Wg.shape[1]
    T = B * S
    NB = T // BLK + E
    P = NB * BLK

    xr = x.reshape(T, D)
    eid = _router(xr, Wg).reshape(T)
    if _DEBUG_JNP_SCHEDULE:
        counts = jnp.sum(eid[:, None] == jnp.arange(E)[None, :], axis=0)
        nb = (counts + BLK - 1) // BLK
        cum = jnp.cumsum(nb)
        start = (cum - nb) * BLK
        total = cum[E - 1]
        order = jnp.argsort(eid, stable=True)
        sc_ = jnp.cumsum(counts) - counts
        es = eid[order]
        slot = start[es] + (jnp.arange(T) - sc_[es])
        src = jnp.full((P,), T, jnp.int32).at[slot].set(
            order.astype(jnp.int32))
        bvec = jnp.arange(NB)
        bc = jnp.minimum(bvec, total - 1)
        be = jnp.sum(cum[None, :] <= bc[:, None], axis=1).astype(jnp.int32)
        lv = (bvec < total).astype(jnp.int32)
    else:
        src, be, lv, _unused = _schedule(eid, E, NB, P)
    opad = _ffn_fused(src, be, lv, xr, W1, b1, W2, b2, NB)
    return opad[:T].reshape(B, S, D)
